# Initial kernel scaffold; baseline (speedup 1.0000x reference)
import jax
import jax.numpy as jnp
from jax.experimental import pallas as pl


def kernel(pred, target):
    raise NotImplementedError("write your pallas kernel here")


# speedup vs baseline: 12.4914x; 12.4914x over previous
import functools

import jax
import jax.numpy as jnp
from jax.experimental import pallas as pl
from jax.experimental.pallas import tpu as pltpu

_OHEM_RATIO = 0.7
_EPS = 1e-07


def _loss_map_body(pred_ref, target_ref, out_ref):
    x = pred_ref[0]
    t = target_ref[0]
    m = jnp.max(x, axis=0)
    s = jnp.sum(jnp.exp(x - m[None]), axis=0)
    tsum = jnp.sum(t, axis=0)
    dot = jnp.sum(t * x, axis=0)
    loss = tsum * (m + jnp.log(s)) - dot
    out_ref[0] = jnp.maximum(loss, 0.0)


def _select_body(loss_ref, out_ref, *, k):
    loss = loss_ref[...]
    bits = loss.view(jnp.int32)

    def step(i, prefix):
        cand = prefix | (jnp.int32(1) << (jnp.int32(30) - i))
        cnt = jnp.sum((bits >= cand).astype(jnp.int32))
        return jnp.where(cnt >= k, cand, prefix)

    vstar = jax.lax.fori_loop(0, 31, step, jnp.int32(0))
    thresh = jax.lax.bitcast_convert_type(vstar, jnp.float32)
    mask = loss >= thresh
    s = jnp.sum(jnp.where(mask, loss, 0.0))
    c = jnp.sum(mask.astype(jnp.int32)).astype(jnp.float32)
    out_ref[...] = jnp.reshape(s / (c + _EPS), (1, 1))


def kernel(pred, target):
    B, C, H, W = pred.shape
    BH = 64
    loss = pl.pallas_call(
        _loss_map_body,
        grid=(B, H // BH),
        in_specs=[
            pl.BlockSpec((1, C, BH, W), lambda b, h: (b, 0, h, 0)),
            pl.BlockSpec((1, C, BH, W), lambda b, h: (b, 0, h, 0)),
        ],
        out_specs=pl.BlockSpec((1, BH, W), lambda b, h: (b, h, 0)),
        out_shape=jax.ShapeDtypeStruct((B, H, W), jnp.float32),
    )(pred, target)

    n = B * H * W
    k = int(n * _OHEM_RATIO)
    flat = loss.reshape(n // 1024, 1024)
    out = pl.pallas_call(
        functools.partial(_select_body, k=k),
        out_shape=jax.ShapeDtypeStruct((1, 1), jnp.float32),
    )(flat)
    return out[0, 0]
